# baseline (speedup 1.0000x reference)
import functools

import jax
import jax.numpy as jnp
from jax import lax
from jax.experimental import pallas as pl
from jax.experimental.pallas import tpu as pltpu
from jax.experimental.pallas import tpu_sc as plsc

VOCAB = 100000
D = 16
C = 15
B = 16384
H = 50

NC = 2
NS = 16
NW = NC * NS
BPW = B // NW
CB = 32
NCHUNK = BPW // CB
ROWS = CB * H

_mesh = plsc.VectorSubcoreMesh(core_axis_name="c", subcore_axis_name="s")


@functools.partial(
    pl.kernel,
    mesh=_mesh,
    compiler_params=pltpu.CompilerParams(use_tc_tiling_on_sc=False),
    out_type=jax.ShapeDtypeStruct((B, D), jnp.float32),
    scratch_types=[
        pltpu.VMEM((BPW * H,), jnp.int32),
        pltpu.VMEM((2, ROWS, D), jnp.float32),
        pltpu.VMEM((CB, D), jnp.float32),
        pltpu.VMEM((D, D), jnp.float32),
        pltpu.VMEM((D,), jnp.float32),
        pltpu.SemaphoreType.DMA,
        pltpu.SemaphoreType.DMA,
    ],
)
def _fused_kernel(x_hbm, tbl_hbm, w_hbm, b_hbm, out_hbm,
                  idx_v, rows_v, fc_v, w_v, b_v, sem0, sem1):
    wid = lax.axis_index("s") * NC + lax.axis_index("c")
    base = wid * BPW
    sems = (sem0, sem1)

    pltpu.sync_copy(w_hbm, w_v)
    pltpu.sync_copy(b_hbm, b_v)
    bv = b_v[...]
    wv = [w_v[d, :] for d in range(D)]

    pltpu.sync_copy(x_hbm.at[pl.ds(base * H, BPW * H)], idx_v)
    pltpu.async_copy(
        tbl_hbm.at[idx_v.at[pl.ds(0, ROWS)]], rows_v.at[0], sems[0]
    )
    for ci in range(NCHUNK):
        cur = ci % 2
        if ci + 1 < NCHUNK:
            pltpu.async_copy(
                tbl_hbm.at[idx_v.at[pl.ds((ci + 1) * ROWS, ROWS)]],
                rows_v.at[1 - cur],
                sems[1 - cur],
            )
        pltpu.make_async_copy(
            tbl_hbm.at[idx_v.at[pl.ds(ci * ROWS, ROWS)]],
            rows_v.at[cur],
            sems[cur],
        ).wait()

        def pool_body(r, cur=cur):
            rv = rows_v.at[cur]
            accs = [rv[r * H + a, :] for a in range(5)]
            for l in range(5, H):
                accs[l % 5] = accs[l % 5] + rv[r * H + l, :]
            p = (accs[0] + accs[1]) + (accs[2] + accs[3]) + accs[4]
            f0 = bv + p[0] * wv[0]
            f1 = p[1] * wv[1]
            f2 = p[2] * wv[2]
            f3 = p[3] * wv[3]
            for d in range(4, D, 4):
                f0 = f0 + p[d] * wv[d]
                f1 = f1 + p[d + 1] * wv[d + 1]
                f2 = f2 + p[d + 2] * wv[d + 2]
                f3 = f3 + p[d + 3] * wv[d + 3]
            fc_v[r, :] = (f0 + f1) + (f2 + f3)

        plsc.parallel_loop(0, CB, 1, unroll=2)(pool_body)
        pltpu.sync_copy(fc_v, out_hbm.at[pl.ds(base + ci * CB, CB)])


@jax.jit
def kernel(x, table, W, b):
    Wp = jnp.zeros((D, D), jnp.float32).at[:, :C].set(W * (1.0 / H))
    b16 = jnp.zeros((D,), jnp.float32).at[:C].set(b)
    return _fused_kernel(x.reshape(-1).astype(jnp.int32), table, Wp, b16)[:, :C]


# speedup vs baseline: 1.0293x; 1.0293x over previous
import functools

import jax
import jax.numpy as jnp
from jax import lax
from jax.experimental import pallas as pl
from jax.experimental.pallas import tpu as pltpu
from jax.experimental.pallas import tpu_sc as plsc

VOCAB = 100000
D = 16
C = 15
B = 16384
H = 50

NC = 2
NS = 16
NW = NC * NS
BPW = B // NW
CB = 32
NCHUNK = BPW // CB
ROWS = CB * H

_mesh = plsc.VectorSubcoreMesh(core_axis_name="c", subcore_axis_name="s")


@functools.partial(
    pl.kernel,
    mesh=_mesh,
    compiler_params=pltpu.CompilerParams(use_tc_tiling_on_sc=False),
    out_type=jax.ShapeDtypeStruct((B, D), jnp.float32),
    scratch_types=[
        pltpu.VMEM((BPW * H,), jnp.int32),
        pltpu.VMEM((3, ROWS, D), jnp.float32),
        pltpu.VMEM((2, CB, D), jnp.float32),
        pltpu.VMEM((D, D), jnp.float32),
        pltpu.VMEM((D,), jnp.float32),
        pltpu.SemaphoreType.DMA,
        pltpu.SemaphoreType.DMA,
        pltpu.SemaphoreType.DMA,
        pltpu.SemaphoreType.DMA,
    ],
)
def _fused_kernel(x_hbm, tbl_hbm, w_hbm, b_hbm, out_hbm,
                  idx_v, rows_v, fc_v, w_v, b_v, sem0, sem1, sem2, osem):
    wid = lax.axis_index("s") * NC + lax.axis_index("c")
    base = wid * BPW
    sems = (sem0, sem1, sem2)

    pltpu.sync_copy(w_hbm, w_v)
    pltpu.sync_copy(b_hbm, b_v)
    bv = b_v[...]
    wv = [w_v[d, :] for d in range(D)]

    pltpu.sync_copy(x_hbm.at[pl.ds(base * H, BPW * H)], idx_v)

    def gather(ci):
        return pltpu.make_async_copy(
            tbl_hbm.at[idx_v.at[pl.ds(ci * ROWS, ROWS)]],
            rows_v.at[ci % 3],
            sems[ci % 3],
        )

    def out_copy(ci):
        return pltpu.make_async_copy(
            fc_v.at[ci % 2], out_hbm.at[pl.ds(base + ci * CB, CB)], osem
        )

    gather(0).start()
    gather(1).start()
    for ci in range(NCHUNK):
        cur = ci % 2
        if ci + 2 < NCHUNK:
            gather(ci + 2).start()
        gather(ci).wait()
        if ci >= 2:
            out_copy(ci - 2).wait()

        def pool_body(r, ci=ci, cur=cur):
            rv = rows_v.at[ci % 3]
            accs = [rv[r * H + a, :] for a in range(5)]
            for l in range(5, H):
                accs[l % 5] = accs[l % 5] + rv[r * H + l, :]
            p = (accs[0] + accs[1]) + (accs[2] + accs[3]) + accs[4]
            f0 = bv + p[0] * wv[0]
            f1 = p[1] * wv[1]
            f2 = p[2] * wv[2]
            f3 = p[3] * wv[3]
            for d in range(4, D, 4):
                f0 = f0 + p[d] * wv[d]
                f1 = f1 + p[d + 1] * wv[d + 1]
                f2 = f2 + p[d + 2] * wv[d + 2]
                f3 = f3 + p[d + 3] * wv[d + 3]
            fc_v[cur, r, :] = (f0 + f1) + (f2 + f3)

        plsc.parallel_loop(0, CB, 1)(pool_body)
        out_copy(ci).start()
    out_copy(NCHUNK - 2).wait()
    out_copy(NCHUNK - 1).wait()


@jax.jit
def kernel(x, table, W, b):
    Wp = jnp.zeros((D, D), jnp.float32).at[:, :C].set(W * (1.0 / H))
    b16 = jnp.zeros((D,), jnp.float32).at[:C].set(b)
    return _fused_kernel(x.reshape(-1).astype(jnp.int32), table, Wp, b16)[:, :C]
